# baseline (speedup 1.0000x reference)
import functools

import jax
import jax.numpy as jnp
from jax import lax
from jax.experimental import pallas as pl
from jax.experimental.pallas import tpu as pltpu
from jax.experimental.pallas import tpu_sc as plsc

_NC = 2
_NS = 16
_NW = _NC * _NS
_L = 16
_ROW = 128


def _cdiv(a, b):
    return (a + b - 1) // b


def kernel(x, edge_index, W, b):
    N, D = x.shape
    E = edge_index.shape[1]

    EPW = _cdiv(_cdiv(E, _NW), _ROW) * _ROW
    N_PAD = _cdiv(N + 1, _ROW) * _ROW
    R = N_PAD // _ROW

    src = edge_index[0].astype(jnp.int32)
    dst = edge_index[1].astype(jnp.int32)
    pad = _NW * EPW - E
    src_p = jnp.concatenate([src, jnp.zeros((pad,), jnp.int32)])
    dst_p = jnp.concatenate([dst, jnp.full((pad,), N, jnp.int32)])
    src3 = src_p.reshape(_NC, _NS, EPW)
    dst3 = dst_p.reshape(_NC, _NS, EPW)

    mesh = plsc.VectorSubcoreMesh(core_axis_name="c", subcore_axis_name="s")

    @functools.partial(
        pl.kernel,
        out_type=jax.ShapeDtypeStruct((_NW, N_PAD), jnp.float32),
        mesh=mesh,
        scratch_types=[
            pltpu.VMEM((EPW,), jnp.int32),
            pltpu.VMEM((N_PAD,), jnp.float32),
            pltpu.VMEM((_L,), jnp.float32),
            pltpu.SemaphoreType.DMA,
        ],
        compiler_params=pltpu.CompilerParams(needs_layout_passes=False),
    )
    def _deg(dst_hbm, cnt_hbm, idx_v, acc_v, ones_v, sem):
        c = lax.axis_index("c")
        s = lax.axis_index("s")
        w = c * _NS + s
        cp = pltpu.async_copy(dst_hbm.at[c, s], idx_v, sem)

        def fz(i, carry):
            acc_v[pl.ds(i * _L, _L)] = jnp.zeros((_L,), jnp.float32)
            return carry

        lax.fori_loop(0, N_PAD // _L, fz, 0)
        ones_v[...] = jnp.full((_L,), 1.0, jnp.float32)
        cp.wait()

        def body(i, carry):
            idx = idx_v[pl.ds(i * _L, _L)]
            plsc.addupdate_scatter(acc_v, [idx], ones_v[...])
            return carry

        lax.fori_loop(0, EPW // _L, body, 0)
        pltpu.sync_copy(acc_v, cnt_hbm.at[w])

    cnt = _deg(dst3)

    x_pad = jnp.concatenate(
        [x, jnp.zeros((N_PAD - N, D), jnp.float32)], axis=0)
    cnt3 = cnt.reshape(_NW, R, _ROW)

    def _tc_pre(x_ref, w_ref, cnt_ref, dis_ref, g_ref):
        h = jnp.dot(x_ref[...], w_ref[...],
                    preferred_element_type=jnp.float32)
        deg = jnp.sum(cnt_ref[...], axis=0) + 1.0
        dis = lax.rsqrt(deg)
        g_ref[...] = h.reshape(R, _ROW) * dis
        dis_ref[...] = dis

    dis2, g2 = pl.pallas_call(
        _tc_pre,
        out_shape=(
            jax.ShapeDtypeStruct((R, _ROW), jnp.float32),
            jax.ShapeDtypeStruct((R, _ROW), jnp.float32),
        ),
    )(x_pad, W, cnt3)
    g_flat = g2.reshape(N_PAD)

    @functools.partial(
        pl.kernel,
        out_type=jax.ShapeDtypeStruct((_NW, N_PAD), jnp.float32),
        mesh=mesh,
        scratch_types=[
            pltpu.VMEM((EPW,), jnp.int32),
            pltpu.VMEM((EPW,), jnp.int32),
            pltpu.VMEM((N_PAD,), jnp.float32),
            pltpu.VMEM((N_PAD,), jnp.float32),
            pltpu.SemaphoreType.DMA,
        ],
        compiler_params=pltpu.CompilerParams(needs_layout_passes=False),
    )
    def _agg(src_hbm, dst_hbm, g_hbm, q_hbm,
             sidx_v, didx_v, g_v, acc_v, sem):
        c = lax.axis_index("c")
        s = lax.axis_index("s")
        w = c * _NS + s
        cp1 = pltpu.async_copy(src_hbm.at[c, s], sidx_v, sem)
        cp2 = pltpu.async_copy(dst_hbm.at[c, s], didx_v, sem)
        cp3 = pltpu.async_copy(g_hbm, g_v, sem)

        def fz(i, carry):
            acc_v[pl.ds(i * _L, _L)] = jnp.zeros((_L,), jnp.float32)
            return carry

        lax.fori_loop(0, N_PAD // _L, fz, 0)
        cp1.wait()
        cp2.wait()
        cp3.wait()

        def body(i, carry):
            sidx = sidx_v[pl.ds(i * _L, _L)]
            didx = didx_v[pl.ds(i * _L, _L)]
            vals = plsc.load_gather(g_v, [sidx])
            plsc.addupdate_scatter(acc_v, [didx], vals)
            return carry

        lax.fori_loop(0, EPW // _L, body, 0)
        pltpu.sync_copy(acc_v, q_hbm.at[w])

    q = _agg(src3, dst3, g_flat)
    q3 = q.reshape(_NW, R, _ROW)

    def _tc_post(dis_ref, g_ref, q_ref, b_ref, out_ref):
        tot = g_ref[...] + jnp.sum(q_ref[...], axis=0)
        out_ref[...] = dis_ref[...] * tot + b_ref[0, 0]

    out2 = pl.pallas_call(
        _tc_post,
        out_shape=jax.ShapeDtypeStruct((R, _ROW), jnp.float32),
    )(dis2, g2, q3, b.reshape(1, 1))

    return out2.reshape(N_PAD)[:N, None]


# speedup vs baseline: 1.3446x; 1.3446x over previous
import functools

import jax
import jax.numpy as jnp
from jax import lax
from jax.experimental import pallas as pl
from jax.experimental.pallas import tpu as pltpu
from jax.experimental.pallas import tpu_sc as plsc

_NC = 2
_NS = 16
_NW = _NC * _NS
_L = 16
_UNROLL = 5
_HUNROLL = 25
_HTEC = 6000
_ATEC = 5600
_WBS = 1000


def kernel(x, edge_index, W, b):
    N, D = x.shape
    E = edge_index.shape[1]
    EPW = E // _NW
    heng = EPW - _HTEC
    aeng = EPW - _ATEC
    assert EPW * _NW == E and N % _L == 0 and N == _WBS * 10
    assert _HTEC % (_L * _HUNROLL) == 0 and heng % _L == 0
    assert _ATEC % (_L * _UNROLL) == 0 and aeng % (_L * _UNROLL) == 0

    edges = edge_index.astype(jnp.int32).reshape(2 * E)
    mesh = plsc.VectorSubcoreMesh(core_axis_name="c", subcore_axis_name="s")
    scp = pltpu.CompilerParams(needs_layout_passes=False)

    @functools.partial(
        pl.kernel,
        out_type=jax.ShapeDtypeStruct(((_NW + _NC) * N,), jnp.float32),
        mesh=mesh,
        scratch_types=[
            pltpu.VMEM((_HTEC,), jnp.int32),
            pltpu.VMEM((heng,), jnp.int32),
            pltpu.VMEM((heng,), jnp.float32),
            pltpu.VMEM((N,), jnp.float32),
            pltpu.VMEM((_L,), jnp.float32),
            pltpu.VMEM_SHARED((N,), jnp.float32),
            pltpu.SemaphoreType.DMA,
            pltpu.SemaphoreType.DMA,
        ],
        compiler_params=scp,
    )
    def _deg(edge_hbm, cnt_hbm, idx_v, eidx_v, eones_v, acc_v, ones_v,
             acc_s, sem, sem2):
        c = lax.axis_index("c")
        s = lax.axis_index("s")
        w = c * _NS + s
        base = E + w * EPW
        cp = pltpu.async_copy(edge_hbm.at[pl.ds(base, _HTEC)], idx_v, sem)
        cpe = pltpu.async_copy(edge_hbm.at[pl.ds(base + _HTEC, heng)],
                               eidx_v, sem)

        def fz(i, carry):
            acc_v[pl.ds(i * _L, _L)] = jnp.zeros((_L,), jnp.float32)
            return carry

        lax.fori_loop(0, N // _L, fz, 0)
        ones_v[...] = jnp.full((_L,), 1.0, jnp.float32)

        def fo(i, carry):
            eones_v[pl.ds(i * _L, _L)] = jnp.full((_L,), 1.0, jnp.float32)
            return carry

        lax.fori_loop(0, heng // _L, fo, 0)

        @pl.when(s == 0)
        def _():
            pltpu.sync_copy(acc_v, acc_s)

        plsc.subcore_barrier()
        cpe.wait()
        eng = pltpu.async_copy(eones_v, acc_s.at[eidx_v], sem2, add=True)
        cp.wait()

        def body(i, carry):
            ones = ones_v[...]
            o = i * (_L * _HUNROLL)
            idxs = [idx_v[pl.ds(o + u * _L, _L)] for u in range(_HUNROLL)]
            for idx in idxs:
                plsc.addupdate_scatter(acc_v, [idx], ones)
            return carry

        lax.fori_loop(0, _HTEC // (_L * _HUNROLL), body, 0)
        pltpu.sync_copy(acc_v, cnt_hbm.at[pl.ds(w * N, N)])
        eng.wait()
        plsc.subcore_barrier()

        @pl.when(s < 10)
        def _():
            pltpu.sync_copy(acc_s.at[pl.ds(s * _WBS, _WBS)],
                            acc_v.at[pl.ds(0, _WBS)])
            pltpu.sync_copy(
                acc_v.at[pl.ds(0, _WBS)],
                cnt_hbm.at[pl.ds((_NW + c) * N + s * _WBS, _WBS)])

    cnt = _deg(edges).reshape(_NW + _NC, N)

    def _tc_pre(x_ref, w_ref, cnt_ref, dis_ref, g_ref):
        h = jnp.dot(x_ref[...], w_ref[...],
                    preferred_element_type=jnp.float32)[:, 0]
        deg = jnp.sum(cnt_ref[...], axis=0) + 1.0
        dis = lax.rsqrt(deg)
        g_ref[...] = h * dis
        dis_ref[...] = dis

    dis1, g1 = pl.pallas_call(
        _tc_pre,
        out_shape=(
            jax.ShapeDtypeStruct((N,), jnp.float32),
            jax.ShapeDtypeStruct((N,), jnp.float32),
        ),
    )(x, W, cnt)

    @functools.partial(
        pl.kernel,
        out_type=jax.ShapeDtypeStruct(((_NW + _NC) * N,), jnp.float32),
        mesh=mesh,
        scratch_types=[
            pltpu.VMEM((EPW,), jnp.int32),
            pltpu.VMEM((_ATEC,), jnp.int32),
            pltpu.VMEM((aeng,), jnp.int32),
            pltpu.VMEM((aeng,), jnp.float32),
            pltpu.VMEM((N,), jnp.float32),
            pltpu.VMEM((N,), jnp.float32),
            pltpu.VMEM_SHARED((N,), jnp.float32),
            pltpu.SemaphoreType.DMA,
            pltpu.SemaphoreType.DMA,
        ],
        compiler_params=scp,
    )
    def _agg(edge_hbm, g_hbm, q_hbm, sidx_v, didx_v, edidx_v, evals_v,
             g_v, acc_v, acc_s, sem, sem2):
        c = lax.axis_index("c")
        s = lax.axis_index("s")
        w = c * _NS + s
        base = w * EPW
        cp1 = pltpu.async_copy(edge_hbm.at[pl.ds(base, EPW)], sidx_v, sem)
        cp2 = pltpu.async_copy(edge_hbm.at[pl.ds(E + base, _ATEC)],
                               didx_v, sem)
        cp3 = pltpu.async_copy(edge_hbm.at[pl.ds(E + base + _ATEC, aeng)],
                               edidx_v, sem)
        cp4 = pltpu.async_copy(g_hbm, g_v, sem)

        def fz(i, carry):
            acc_v[pl.ds(i * _L, _L)] = jnp.zeros((_L,), jnp.float32)
            return carry

        lax.fori_loop(0, N // _L, fz, 0)

        @pl.when(s == 0)
        def _():
            pltpu.sync_copy(acc_v, acc_s)

        plsc.subcore_barrier()
        cp1.wait()
        cp4.wait()

        def gbody(i, carry):
            o = i * (_L * _UNROLL)
            sidxs = [sidx_v[pl.ds(_ATEC + o + u * _L, _L)]
                     for u in range(_UNROLL)]
            vals = [plsc.load_gather(g_v, [si]) for si in sidxs]
            for u in range(_UNROLL):
                evals_v[pl.ds(o + u * _L, _L)] = vals[u]
            return carry

        lax.fori_loop(0, aeng // (_L * _UNROLL), gbody, 0)
        cp3.wait()
        eng = pltpu.async_copy(evals_v, acc_s.at[edidx_v], sem2, add=True)
        cp2.wait()

        def body(i, carry):
            o = i * (_L * _UNROLL)
            sidxs = [sidx_v[pl.ds(o + u * _L, _L)] for u in range(_UNROLL)]
            didxs = [didx_v[pl.ds(o + u * _L, _L)] for u in range(_UNROLL)]
            vals = [plsc.load_gather(g_v, [si]) for si in sidxs]
            for di, v in zip(didxs, vals):
                plsc.addupdate_scatter(acc_v, [di], v)
            return carry

        lax.fori_loop(0, _ATEC // (_L * _UNROLL), body, 0)
        pltpu.sync_copy(acc_v, q_hbm.at[pl.ds(w * N, N)])
        eng.wait()
        plsc.subcore_barrier()

        @pl.when(s < 10)
        def _():
            pltpu.sync_copy(acc_s.at[pl.ds(s * _WBS, _WBS)],
                            acc_v.at[pl.ds(0, _WBS)])
            pltpu.sync_copy(
                acc_v.at[pl.ds(0, _WBS)],
                q_hbm.at[pl.ds((_NW + c) * N + s * _WBS, _WBS)])

    q = _agg(edges, g1).reshape(_NW + _NC, N)

    def _tc_post(dis_ref, g_ref, q_ref, b_ref, out_ref):
        tot = g_ref[...] + jnp.sum(q_ref[...], axis=0)
        out_ref[...] = dis_ref[...] * tot + b_ref[0]

    out1 = pl.pallas_call(
        _tc_post,
        out_shape=jax.ShapeDtypeStruct((N,), jnp.float32),
    )(dis1, g1, q, b)

    return out1[:, None]


# speedup vs baseline: 1.6995x; 1.2639x over previous
import functools

import jax
import jax.numpy as jnp
from jax import lax
from jax.experimental import pallas as pl
from jax.experimental.pallas import tpu as pltpu
from jax.experimental.pallas import tpu_sc as plsc

_NC = 2
_NS = 16
_NW = _NC * _NS
_L = 16
_UNROLL = 5
_HUNROLL = 25


def kernel(x, edge_index, W, b):
    N, D = x.shape
    E = edge_index.shape[1]
    EPW = E // _NW
    assert EPW * _NW == E and N % _L == 0
    assert EPW % (_L * _UNROLL) == 0 and EPW % (_L * _HUNROLL) == 0

    edges = edge_index.astype(jnp.int32).reshape(2 * E)
    mesh = plsc.VectorSubcoreMesh(core_axis_name="c", subcore_axis_name="s")
    scp = pltpu.CompilerParams(needs_layout_passes=False)

    @functools.partial(
        pl.kernel,
        out_type=jax.ShapeDtypeStruct((_NW, N), jnp.float32),
        mesh=mesh,
        scratch_types=[
            pltpu.VMEM((EPW,), jnp.int32),
            pltpu.VMEM((N,), jnp.float32),
            pltpu.VMEM((_L,), jnp.float32),
            pltpu.SemaphoreType.DMA,
        ],
        compiler_params=scp,
    )
    def _deg(edge_hbm, cnt_hbm, idx_v, acc_v, ones_v, sem):
        c = lax.axis_index("c")
        s = lax.axis_index("s")
        w = c * _NS + s
        cp = pltpu.async_copy(edge_hbm.at[pl.ds(E + w * EPW, EPW)],
                              idx_v, sem)

        def fz(i, carry):
            acc_v[pl.ds(i * _L, _L)] = jnp.zeros((_L,), jnp.float32)
            return carry

        lax.fori_loop(0, N // _L, fz, 0)
        ones_v[...] = jnp.full((_L,), 1.0, jnp.float32)
        cp.wait()

        def body(i, carry):
            ones = ones_v[...]
            base = i * (_L * _HUNROLL)
            idxs = [idx_v[pl.ds(base + u * _L, _L)] for u in range(_HUNROLL)]
            for idx in idxs:
                plsc.addupdate_scatter(acc_v, [idx], ones)
            return carry

        lax.fori_loop(0, EPW // (_L * _HUNROLL), body, 0)
        pltpu.sync_copy(acc_v, cnt_hbm.at[w])

    def _tc_mv(x_ref, w_ref, h_ref):
        h_ref[...] = jnp.dot(x_ref[...], w_ref[...],
                             preferred_element_type=jnp.float32)[:, 0]

    h1 = pl.pallas_call(
        _tc_mv,
        out_shape=jax.ShapeDtypeStruct((N,), jnp.float32),
    )(x, W)

    cnt = _deg(edges)

    def _tc_pre(h_ref, cnt_ref, dis_ref, g_ref):
        deg = jnp.sum(cnt_ref[...], axis=0) + 1.0
        dis = lax.rsqrt(deg)
        g_ref[...] = h_ref[...] * dis
        dis_ref[...] = dis

    dis1, g1 = pl.pallas_call(
        _tc_pre,
        out_shape=(
            jax.ShapeDtypeStruct((N,), jnp.float32),
            jax.ShapeDtypeStruct((N,), jnp.float32),
        ),
    )(h1, cnt)

    @functools.partial(
        pl.kernel,
        out_type=jax.ShapeDtypeStruct((_NW, N), jnp.float32),
        mesh=mesh,
        scratch_types=[
            pltpu.VMEM((EPW,), jnp.int32),
            pltpu.VMEM((EPW,), jnp.int32),
            pltpu.VMEM((N,), jnp.float32),
            pltpu.VMEM((N,), jnp.float32),
            pltpu.SemaphoreType.DMA,
        ],
        compiler_params=scp,
    )
    def _agg(edge_hbm, g_hbm, q_hbm, sidx_v, didx_v, g_v, acc_v, sem):
        c = lax.axis_index("c")
        s = lax.axis_index("s")
        w = c * _NS + s
        cp1 = pltpu.async_copy(edge_hbm.at[pl.ds(w * EPW, EPW)],
                               sidx_v, sem)
        cp2 = pltpu.async_copy(edge_hbm.at[pl.ds(E + w * EPW, EPW)],
                               didx_v, sem)
        cp3 = pltpu.async_copy(g_hbm, g_v, sem)

        def fz(i, carry):
            acc_v[pl.ds(i * _L, _L)] = jnp.zeros((_L,), jnp.float32)
            return carry

        lax.fori_loop(0, N // _L, fz, 0)
        cp1.wait()
        cp2.wait()
        cp3.wait()

        def body(i, carry):
            base = i * (_L * _UNROLL)
            sidxs = [sidx_v[pl.ds(base + u * _L, _L)]
                     for u in range(_UNROLL)]
            didxs = [didx_v[pl.ds(base + u * _L, _L)]
                     for u in range(_UNROLL)]
            vals = [plsc.load_gather(g_v, [si]) for si in sidxs]
            for di, v in zip(didxs, vals):
                plsc.addupdate_scatter(acc_v, [di], v)
            return carry

        lax.fori_loop(0, EPW // (_L * _UNROLL), body, 0)
        pltpu.sync_copy(acc_v, q_hbm.at[w])

    q = _agg(edges, g1)

    def _tc_post(dis_ref, g_ref, q_ref, b_ref, out_ref):
        tot = g_ref[...] + jnp.sum(q_ref[...], axis=0)
        out_ref[...] = dis_ref[...] * tot + b_ref[0]

    out1 = pl.pallas_call(
        _tc_post,
        out_shape=jax.ShapeDtypeStruct((N,), jnp.float32),
    )(dis1, g1, q, b)

    return out1[:, None]
